# baseline (speedup 1.0000x reference)
import jax
import jax.numpy as jnp
from jax import lax
from jax.experimental import pallas as pl
from jax.experimental.pallas import tpu as pltpu
from jax.experimental.pallas import tpu_sc as plsc

_NT = 60
_PS = 3375
_B = 16384
_NC = 2
_NS = 16
_TG = 4
_BG = 8
_TPG = _NT // _TG
_BPG = _B // _BG
_HALF = _BPG // 2
_STRIPE = _BPG // _TG
_LANES = 16


def _sc_body(x_hbm, W_hbm, out_hbm, tab_v, x_v, acc_v, tmp_v, sem0, sem1,
             sem2, shared):
    c = lax.axis_index("c")
    s = lax.axis_index("s")
    tg = s % _TG
    bg = c * (_NS // _TG) + s // _TG
    sbase = s - tg

    tab_dma = pltpu.async_copy(
        W_hbm.at[pl.ds(tg * _TPG, _TPG), :], tab_v, sem0)
    x_dma0 = pltpu.async_copy(
        x_hbm.at[pl.ds(tg * _TPG, _TPG), pl.ds(bg * _BPG, _HALF)],
        x_v.at[:, pl.ds(0, _HALF)], sem1)
    x_dma1 = pltpu.async_copy(
        x_hbm.at[pl.ds(tg * _TPG, _TPG), pl.ds(bg * _BPG + _HALF, _HALF)],
        x_v.at[:, pl.ds(_HALF, _HALF)], sem2)

    def body(v, _):
        pos = pl.multiple_of(v * _LANES, _LANES)
        acc = jnp.zeros((_LANES,), jnp.float32)
        for k in range(_TPG):
            kvec = jnp.full((_LANES,), k, jnp.int32)
            xv = x_v[k, pl.ds(pos, _LANES)]
            acc = acc + plsc.load_gather(tab_v, [kvec, xv])
        acc_v[pl.ds(pos, _LANES)] = acc
        return 0

    tab_dma.wait()
    x_dma0.wait()
    lax.fori_loop(0, _HALF // _LANES, body, 0)
    x_dma1.wait()
    lax.fori_loop(_HALF // _LANES, _BPG // _LANES, body, 0)

    pltpu.sync_copy(acc_v, shared.at[s])
    plsc.subcore_barrier()

    for j in range(_TG):
        pltpu.sync_copy(shared.at[sbase + j, pl.ds(tg * _STRIPE, _STRIPE)],
                        tmp_v.at[j])

    def red2(v, _):
        pos = pl.multiple_of(v * _LANES, _LANES)
        tot = jnp.zeros((_LANES,), jnp.float32)
        for j in range(_TG):
            tot = tot + tmp_v[j, pl.ds(pos, _LANES)]
        acc_v[pl.ds(pos, _LANES)] = tot
        return 0

    lax.fori_loop(0, _STRIPE // _LANES, red2, 0)
    pltpu.sync_copy(acc_v.at[pl.ds(0, _STRIPE)],
                    out_hbm.at[pl.ds(bg * _BPG + tg * _STRIPE, _STRIPE)])


@jax.jit
def _sc_call(xT, W2):
    mesh = plsc.VectorSubcoreMesh(
        core_axis_name="c", subcore_axis_name="s",
        num_cores=_NC, num_subcores=_NS)
    f = pl.kernel(
        _sc_body,
        out_type=jax.ShapeDtypeStruct((_B,), jnp.float32),
        mesh=mesh,
        scratch_types=[
            pltpu.VMEM((_TPG, _PS), jnp.float32),
            pltpu.VMEM((_TPG, _BPG), jnp.int32),
            pltpu.VMEM((_BPG,), jnp.float32),
            pltpu.VMEM((_TG, _STRIPE), jnp.float32),
            pltpu.SemaphoreType.DMA,
            pltpu.SemaphoreType.DMA,
            pltpu.SemaphoreType.DMA,
            pltpu.VMEM_SHARED((_NS, _BPG), jnp.float32),
        ],
        compiler_params=pltpu.CompilerParams(
            use_tc_tiling_on_sc=False, needs_layout_passes=False),
    )
    return f(xT, W2)


_TBLK = 1024


def _transpose_body(x_ref, o_ref):
    o_ref[...] = x_ref[...].T


@jax.jit
def _tc_transpose(x):
    return pl.pallas_call(
        _transpose_body,
        grid=(_B // _TBLK,),
        in_specs=[pl.BlockSpec((_TBLK, _NT), lambda i: (i, 0))],
        out_specs=pl.BlockSpec((_NT, _TBLK), lambda i: (0, i)),
        out_shape=jax.ShapeDtypeStruct((_NT, _B), jnp.int32),
    )(x)


def kernel(x, W):
    W2 = W.reshape(_NT, _PS)
    xT = _tc_transpose(x.astype(jnp.int32))
    y = _sc_call(xT, W2)
    return y[:, None]


# speedup vs baseline: 1.6629x; 1.6629x over previous
import jax
import jax.numpy as jnp
from jax import lax
from jax.experimental import pallas as pl
from jax.experimental.pallas import tpu as pltpu
from jax.experimental.pallas import tpu_sc as plsc

_NT = 60
_PS = 3375
_B = 16384
_NC = 2
_NS = 16
_TG = 8
_BG = 4
_TPG = 8
_BPG = _B // _BG
_HALF = _BPG // 2
_STRIPE = _BPG // _TG
_LANES = 16


def _sc_body(x_hbm, W_hbm, out_hbm, tab_v, x_v, acc_v, tmp_v, sem0, sem1,
             sem2, shared):
    c = lax.axis_index("c")
    s = lax.axis_index("s")
    tg = s % _TG
    bg = c * (_NS // _TG) + s // _TG
    sbase = s - tg
    off = jnp.where(tg == _TG - 1, _NT - _TPG, tg * _TPG)
    nown = jnp.where(tg == _TG - 2, _NT - (_TG - 1) * _TPG, _TPG)

    tab_dma = pltpu.async_copy(W_hbm.at[pl.ds(off, _TPG), :], tab_v, sem0)
    x_dma0 = pltpu.async_copy(
        x_hbm.at[pl.ds(off, _TPG), pl.ds(bg * _BPG, _HALF)],
        x_v.at[:, pl.ds(0, _HALF)], sem1)
    x_dma1 = pltpu.async_copy(
        x_hbm.at[pl.ds(off, _TPG), pl.ds(bg * _BPG + _HALF, _HALF)],
        x_v.at[:, pl.ds(_HALF, _HALF)], sem2)

    def body(v, _):
        pos = pl.multiple_of(v * _LANES, _LANES)
        acc = jnp.zeros((_LANES,), jnp.float32)
        for k in range(_TPG):
            kvec = jnp.full((_LANES,), k, jnp.int32)
            xv = x_v[k, pl.ds(pos, _LANES)]
            val = plsc.load_gather(tab_v, [kvec, xv])
            acc = acc + jnp.where(k < nown, val, 0.0)
        acc_v[pl.ds(pos, _LANES)] = acc
        return 0

    tab_dma.wait()
    x_dma0.wait()
    lax.fori_loop(0, _HALF // _LANES, body, 0)
    x_dma1.wait()
    lax.fori_loop(_HALF // _LANES, _BPG // _LANES, body, 0)

    pltpu.sync_copy(acc_v, shared.at[s])
    plsc.subcore_barrier()

    for j in range(_TG):
        pltpu.sync_copy(shared.at[sbase + j, pl.ds(tg * _STRIPE, _STRIPE)],
                        tmp_v.at[j])

    def red(v, _):
        pos = pl.multiple_of(v * _LANES, _LANES)
        tot = jnp.zeros((_LANES,), jnp.float32)
        for j in range(_TG):
            tot = tot + tmp_v[j, pl.ds(pos, _LANES)]
        acc_v[pl.ds(pos, _LANES)] = tot
        return 0

    lax.fori_loop(0, _STRIPE // _LANES, red, 0)
    pltpu.sync_copy(acc_v.at[pl.ds(0, _STRIPE)],
                    out_hbm.at[pl.ds(bg * _BPG + tg * _STRIPE, _STRIPE)])


@jax.jit
def _sc_call(xT, W2):
    mesh = plsc.VectorSubcoreMesh(
        core_axis_name="c", subcore_axis_name="s",
        num_cores=_NC, num_subcores=_NS)
    f = pl.kernel(
        _sc_body,
        out_type=jax.ShapeDtypeStruct((_B,), jnp.float32),
        mesh=mesh,
        scratch_types=[
            pltpu.VMEM((_TPG, _PS), jnp.float32),
            pltpu.VMEM((_TPG, _BPG), jnp.int32),
            pltpu.VMEM((_BPG,), jnp.float32),
            pltpu.VMEM((_TG, _STRIPE), jnp.float32),
            pltpu.SemaphoreType.DMA,
            pltpu.SemaphoreType.DMA,
            pltpu.SemaphoreType.DMA,
            pltpu.VMEM_SHARED((_NS, _BPG), jnp.float32),
        ],
        compiler_params=pltpu.CompilerParams(
            use_tc_tiling_on_sc=False, needs_layout_passes=False),
    )
    return f(xT, W2)


def kernel(x, W):
    W2 = W.reshape(_NT, _PS)
    xT = x.T.astype(jnp.int32)
    y = _sc_call(xT, W2)
    return y[:, None]
